# baseline (speedup 1.0000x reference)
import functools

import jax
import jax.numpy as jnp
from jax import lax
from jax.experimental import pallas as pl
from jax.experimental.pallas import tpu as pltpu
from jax.experimental.pallas import tpu_sc as plsc

N = 10000
E = 160000
NP = 10240
NDUMP = 10240
NQ = NP // 4
NACC = NQ + 128
NC = 2
NS = 16
NPROD = NC * NS
EPT = 5120
EPAD = 163840
NBD = EPAD // (NS * 128)
SB = 128
SBLOG = 7
NSEG = EPT // SB
GS = 4
QS = SB // GS
W = 256
ROWS_PER_TILE = NP // NS
ACC_PER_TILE = NACC // NS
WB_PER_TILE = NQ // NS
BN = 1024
GRID = NP // BN

_mesh = plsc.VectorSubcoreMesh(core_axis_name="c", subcore_axis_name="s")


@functools.partial(
    pl.kernel,
    out_type=jax.ShapeDtypeStruct((NC, NP), jnp.float32),
    mesh=_mesh,
    scratch_types=[
        pltpu.VMEM((NBD, 128), jnp.int32),
        pltpu.VMEM((128,), jnp.float32),
        pltpu.VMEM_SHARED((NP,), jnp.float32),
    ],
)
def _deg_kernel(idx_hbm, zeros_hbm, ones_hbm, out_hbm, idx_v, ones_v, deg_sp):
    c = lax.axis_index("c")
    s = lax.axis_index("s")
    pltpu.sync_copy(ones_hbm, ones_v)
    pltpu.sync_copy(zeros_hbm.at[pl.ds(0, ROWS_PER_TILE)],
                    deg_sp.at[pl.ds(s * ROWS_PER_TILE, ROWS_PER_TILE)])
    pltpu.sync_copy(idx_hbm.at[c, s], idx_v)
    plsc.subcore_barrier()

    def body(b, carry):
        pltpu.sync_copy(ones_v, deg_sp.at[idx_v.at[b]], add=True)
        return carry

    lax.fori_loop(0, NBD, body, None)
    plsc.subcore_barrier()

    @pl.when(s == 0)
    def _():
        pltpu.sync_copy(deg_sp, out_hbm.at[c])


@functools.partial(
    pl.kernel,
    out_type=[
        jax.ShapeDtypeStruct((4, NPROD, NSEG, SB), jnp.int32),
        jax.ShapeDtypeStruct((4, NPROD, NSEG, SB), jnp.int32),
        jax.ShapeDtypeStruct((NC, NS, 4, 16), jnp.int32),
    ],
    mesh=_mesh,
    scratch_types=[
        pltpu.VMEM((EPT // 16, 16), jnp.int32),
        pltpu.VMEM((EPT // 16, 16), jnp.int32),
        pltpu.VMEM((4, NSEG, SB), jnp.int32),
        pltpu.VMEM((4, NSEG, SB), jnp.int32),
        pltpu.VMEM((4, 16), jnp.int32),
    ],
    compiler_params=pltpu.CompilerParams(needs_layout_passes=False),
)
def _part_kernel(src_hbm, dst_hbm, fill_hbm, srcl_hbm, dstl_hbm, cnt_hbm,
                 src_v, dst_v, csrc_v, cdst_v, cnt_v):
    c = lax.axis_index("c")
    s = lax.axis_index("s")
    p = c * NS + s
    pltpu.sync_copy(src_hbm.at[c, s], src_v)
    pltpu.sync_copy(dst_hbm.at[c, s], dst_v)
    for side in range(4):
        pltpu.sync_copy(fill_hbm.at[0], csrc_v.at[side])
        pltpu.sync_copy(fill_hbm.at[1], cdst_v.at[side])

    zero16 = jnp.zeros((16,), jnp.int32)
    iota16 = lax.iota(jnp.int32, 16)

    def _psum16(v):
        for k in (1, 2, 4, 8):
            sh = v.at[jnp.maximum(iota16 - k, 0)].get(mode="promise_in_bounds")
            v = v + jnp.where(iota16 >= k, sh, 0)
        return v

    tot = jnp.full((16,), 15, jnp.int32)

    def body(i, offs):
        s16 = src_v[i]
        d16 = dst_v[i]
        new_offs = []
        for q in range(4):
            mq = jnp.logical_and(d16 >= q * NQ, d16 < (q + 1) * NQ) \
                if q < 3 else (d16 >= 3 * NQ)
            dloc = d16 - q * NQ
            csum = _psum16(jnp.where(mq, 1, 0))
            pos = offs[q] + csum - 1
            rq, cq = pos >> SBLOG, pos & (SB - 1)
            plsc.store_scatter(csrc_v.at[q], [rq, cq], s16, mask=mq)
            plsc.store_scatter(cdst_v.at[q], [rq, cq], dloc, mask=mq)
            new_offs.append(
                offs[q] + csum.at[tot].get(mode="promise_in_bounds"))
        return tuple(new_offs)

    offs = lax.fori_loop(0, EPT // 16, body,
                         (zero16, zero16, zero16, zero16))
    for q in range(4):
        cnt_v[q] = offs[q]
    for side in range(4):
        pltpu.sync_copy(csrc_v.at[side], srcl_hbm.at[side, p])
        pltpu.sync_copy(cdst_v.at[side], dstl_hbm.at[side, p])
    pltpu.sync_copy(cnt_v, cnt_hbm.at[c, s])


def _make_agg_kernel(R):
    scratch = [
        pltpu.VMEM((24, SB), jnp.int32),
        pltpu.VMEM((24, SB), jnp.int32),
        pltpu.VMEM((4, 16), jnp.int32),
        pltpu.VMEM((2, SB, 2, 128), jnp.float32),
        pltpu.VMEM_SHARED((NACC, 2, 128), jnp.float32),
        pltpu.SemaphoreType.DMA,
        pltpu.SemaphoreType.DMA,
    ]

    @functools.partial(
        pl.kernel,
        out_type=jax.ShapeDtypeStruct((R, NC, 2, NQ, 2, 128), jnp.float32),
        mesh=_mesh,
        scratch_types=scratch,
    )
    def agg(*refs):
        tables = refs[:R]
        srcl_hbm, dstl_hbm, cnt_hbm, zeros_hbm, out_hbm = refs[R:R + 5]
        src_v, dst_v, cnt_v, rows_v, agg_sp, gsem, ssem = refs[R + 5:]
        c = lax.axis_index("c")
        s = lax.axis_index("s")
        for r in range(R):
          table = tables[r]
          for k2 in range(2):
            q_idx = 2 * c + k2
            pltpu.sync_copy(
                zeros_hbm,
                agg_sp.at[pl.ds(s * ACC_PER_TILE, ACC_PER_TILE)])
            plsc.subcore_barrier()
            for k in range(2):
                p = 2 * s + k
                pltpu.sync_copy(cnt_hbm.at[p // NS, lax.rem(p, NS)], cnt_v)
                cnt = jnp.where(c == 0, cnt_v[k2][0], cnt_v[k2 + 2][0])
                nb_all = lax.div(cnt + (SB - 1), SB)
                for off, sz in ((0, 24), (24, 16)):
                  nb = lax.max(0, lax.min(nb_all - off, sz))

                  @pl.when(nb > 0)
                  def _(table=table, nb=nb, p=p, off=off, sz=sz,
                        q_idx=q_idx):
                    pltpu.sync_copy(
                        srcl_hbm.at[q_idx, p, pl.ds(off, sz)],
                        src_v.at[pl.ds(0, sz)])
                    pltpu.sync_copy(
                        dstl_hbm.at[q_idx, p, pl.ds(off, sz)],
                        dst_v.at[pl.ds(0, sz)])
                    for g in range(GS):
                        pltpu.async_copy(
                            table.at[src_v.at[0, pl.ds(g * QS, QS)]],
                            rows_v.at[0, pl.ds(g * QS, QS)], gsem)

                    def body(b, carry, table=table, nb=nb):
                        j = lax.rem(b, 2)
                        jn = lax.rem(b + 1, 2)

                        @pl.when(b > 0)
                        def _():
                            pltpu.make_async_copy(
                                rows_v.at[jn],
                                agg_sp.at[dst_v.at[b - 1]], ssem).wait()

                        @pl.when(b + 1 < nb)
                        def _():
                            for g in range(GS):
                                pltpu.async_copy(
                                    table.at[
                                        src_v.at[b + 1, pl.ds(g * QS, QS)]],
                                    rows_v.at[jn, pl.ds(g * QS, QS)], gsem)

                        pltpu.make_async_copy(
                            table.at[src_v.at[b]], rows_v.at[j], gsem).wait()
                        pltpu.async_copy(
                            rows_v.at[j], agg_sp.at[dst_v.at[b]], ssem,
                            add=True)
                        return carry

                    lax.fori_loop(0, nb, body, None)
                    pltpu.make_async_copy(
                        rows_v.at[lax.rem(nb - 1, 2)],
                        agg_sp.at[dst_v.at[nb - 1]], ssem).wait()

            plsc.subcore_barrier()
            pltpu.sync_copy(
                agg_sp.at[pl.ds(s * WB_PER_TILE, WB_PER_TILE)],
                out_hbm.at[r, c, k2, pl.ds(s * WB_PER_TILE, WB_PER_TILE)])
            plsc.subcore_barrier()

    return agg


_agg1 = _make_agg_kernel(1)
_agg2 = _make_agg_kernel(2)


def _norm(deg_blk):
    return lax.rsqrt(jnp.maximum(deg_blk, 1.0))


def _pre_body(x_ref, dout_ref, out_ref):
    xs = x_ref[...] * _norm(dout_ref[...])
    out_ref[:, 0, :] = xs[:, :128]
    out_ref[:, 1, :] = xs[:, 128:]


def _mm1_body(agg_ref, din_ref, dout_ref, w_ref, b_ref, out_ref):
    a = agg_ref[...] * _norm(din_ref[...])
    h = jnp.dot(a, w_ref[...], preferred_element_type=jnp.float32) + b_ref[...]
    h = jnp.maximum(h, 0.0) * _norm(dout_ref[...])
    for j in range(2):
        for t in range(2):
            out_ref[j, :, t, :] = h[:, (2 * j + t) * 128:(2 * j + t + 1) * 128]


def _mm2_body(agg_ref, din_ref, wc2_ref, bc2_ref, wl1_ref, bl1_ref, wo_ref,
              bo_ref, out_ref):
    a = jnp.concatenate([agg_ref[0], agg_ref[1]], axis=1)
    a = a * _norm(din_ref[...])
    h = jnp.dot(a, wc2_ref[...], preferred_element_type=jnp.float32)
    h = jnp.maximum(h + bc2_ref[...], 0.0)
    h = jnp.dot(h, wl1_ref[...], preferred_element_type=jnp.float32)
    h = jnp.maximum(h + bl1_ref[...], 0.0)
    out_ref[...] = (jnp.dot(h, wo_ref[...], preferred_element_type=jnp.float32)
                    + bo_ref[...])


def _full(shape):
    return pl.BlockSpec(shape, lambda i: tuple(0 for _ in shape))


_pre_call = pl.pallas_call(
    _pre_body,
    grid=(GRID,),
    in_specs=[
        pl.BlockSpec((BN, 256), lambda i: (i, 0)),
        pl.BlockSpec((BN, 1), lambda i: (i, 0)),
    ],
    out_specs=pl.BlockSpec((BN, 2, 128), lambda i: (i, 0, 0)),
    out_shape=jax.ShapeDtypeStruct((NP, 2, 128), jnp.float32),
)

_mm1_call = pl.pallas_call(
    _mm1_body,
    grid=(GRID,),
    in_specs=[
        pl.BlockSpec((BN, W), lambda i: (i, 0)),
        pl.BlockSpec((BN, 1), lambda i: (i, 0)),
        pl.BlockSpec((BN, 1), lambda i: (i, 0)),
        _full((256, 512)),
        _full((1, 512)),
    ],
    out_specs=pl.BlockSpec((2, BN, 2, 128), lambda i: (0, i, 0, 0)),
    out_shape=jax.ShapeDtypeStruct((2, NP, 2, 128), jnp.float32),
)

_mm2_call = pl.pallas_call(
    _mm2_body,
    grid=(GRID,),
    in_specs=[
        pl.BlockSpec((2, BN, W), lambda i: (0, i, 0)),
        pl.BlockSpec((BN, 1), lambda i: (i, 0)),
        _full((512, 512)),
        _full((1, 512)),
        _full((512, 512)),
        _full((1, 512)),
        _full((512, 128)),
        _full((1, 128)),
    ],
    out_specs=pl.BlockSpec((BN, 128), lambda i: (i, 0)),
    out_shape=jax.ShapeDtypeStruct((NP, 128), jnp.float32),
)


def kernel(x, edge_index, Wc1, bc1, Wc2, bc2, Wl1, bl1, Wo, bo):
    src = edge_index[0].astype(jnp.int32)
    dst = edge_index[1].astype(jnp.int32)
    pad = EPAD - E
    src_g = jnp.concatenate([src, jnp.zeros((pad,), jnp.int32)])
    dst_g = jnp.concatenate([dst, jnp.full((pad,), NDUMP, jnp.int32)])
    dst_p = jnp.concatenate([dst, jnp.full((pad,), N, jnp.int32)])
    src_d = jnp.concatenate([src, jnp.full((pad,), N, jnp.int32)])

    src_part = src_g.reshape(NC, NS, EPT // 16, 16)
    dst_part = dst_g.reshape(NC, NS, EPT // 16, 16)
    deg_idx = jnp.stack([src_d, dst_p]).reshape(NC, NS, NBD, 128)

    zeros_flat = jnp.zeros((ROWS_PER_TILE,), jnp.float32)
    zeros_w = jnp.zeros((ACC_PER_TILE, 2, 128), jnp.float32)
    ones = jnp.ones((128,), jnp.float32)
    fill = jnp.stack([jnp.zeros((EPT,), jnp.int32),
                      jnp.full((EPT,), NQ, jnp.int32)]).reshape(2, NSEG, SB)

    degs = _deg_kernel(deg_idx, zeros_flat, ones)
    deg_out = degs[0].reshape(NP, 1)
    deg_in = degs[1].reshape(NP, 1)

    srcl, dstl, cnts = _part_kernel(src_part, dst_part, fill)

    x_pad = jnp.pad(x, ((0, NP - N), (0, 0)))

    table1 = _pre_call(x_pad, deg_out)
    agg1 = _agg1(table1, srcl, dstl, cnts, zeros_w)
    h1s = _mm1_call(agg1.reshape(NP, W), deg_in, deg_out,
                    Wc1, bc1.reshape(1, 512))

    agg2 = _agg2(h1s[0], h1s[1], srcl, dstl, cnts, zeros_w)
    out = _mm2_call(agg2.reshape(2, NP, W), deg_in,
                    Wc2, bc2.reshape(1, 512),
                    Wl1, bl1.reshape(1, 512),
                    Wo, bo.reshape(1, 128))
    return out[:N]


# speedup vs baseline: 2.1671x; 2.1671x over previous
import functools

import jax
import jax.numpy as jnp
from jax import lax
from jax.experimental import pallas as pl
from jax.experimental.pallas import tpu as pltpu
from jax.experimental.pallas import tpu_sc as plsc

N = 10000
E = 160000
NP = 10240
NDUMP = N
NC = 2
NS = 16
EPAD = 163840
NBATCH = EPAD // (NS * 128)
ROWS_PER_TILE = NP // NS
BN = 1024
GRID = NP // BN

_mesh = plsc.VectorSubcoreMesh(core_axis_name="c", subcore_axis_name="s")


@functools.partial(
    pl.kernel,
    out_type=jax.ShapeDtypeStruct((NC, NP), jnp.float32),
    mesh=_mesh,
    scratch_types=[
        pltpu.VMEM((NBATCH, 128), jnp.int32),
        pltpu.VMEM((128,), jnp.float32),
        pltpu.VMEM_SHARED((NP,), jnp.float32),
    ],
)
def _deg_kernel(idx_hbm, zeros_hbm, ones_hbm, out_hbm, idx_v, ones_v, deg_sp):
    c = lax.axis_index("c")
    s = lax.axis_index("s")
    pltpu.sync_copy(ones_hbm, ones_v)
    pltpu.sync_copy(zeros_hbm.at[pl.ds(0, ROWS_PER_TILE)],
                    deg_sp.at[pl.ds(s * ROWS_PER_TILE, ROWS_PER_TILE)])
    pltpu.sync_copy(idx_hbm.at[c, s], idx_v)
    plsc.subcore_barrier()

    def body(b, carry):
        pltpu.sync_copy(ones_v, deg_sp.at[idx_v.at[b]], add=True)
        return carry

    lax.fori_loop(0, NBATCH, body, None)
    plsc.subcore_barrier()

    @pl.when(s == 0)
    def _():
        pltpu.sync_copy(deg_sp, out_hbm.at[c])


HB = NBATCH // 2
GS = 4
QS = 128 // GS


def _make_agg_kernel(R):
    scratch = [
        pltpu.VMEM((HB, 128), jnp.int32),
        pltpu.VMEM((HB, 128), jnp.int32),
        pltpu.VMEM((2, 128, 128), jnp.float32),
        pltpu.VMEM_SHARED((NP, 128), jnp.float32),
        pltpu.SemaphoreType.DMA,
        pltpu.SemaphoreType.DMA,
    ]

    @functools.partial(
        pl.kernel,
        out_type=jax.ShapeDtypeStruct((R, NC, NP, 128), jnp.float32),
        mesh=_mesh,
        scratch_types=scratch,
    )
    def agg(*refs):
        tables = refs[:R]
        src_hbm, dst_hbm, zeros_hbm, out_hbm = refs[R:R + 4]
        src_v, dst_v, rows_v, agg_sp, gsem, ssem = refs[R + 4:]
        c = lax.axis_index("c")
        s = lax.axis_index("s")
        for r in range(R):
            pltpu.sync_copy(
                zeros_hbm,
                agg_sp.at[pl.ds(s * ROWS_PER_TILE, ROWS_PER_TILE)])
            plsc.subcore_barrier()
            table = tables[r]
            for h in range(NBATCH // HB):
                pltpu.sync_copy(src_hbm.at[c, s, pl.ds(h * HB, HB)], src_v)
                pltpu.sync_copy(dst_hbm.at[s, pl.ds(h * HB, HB)], dst_v)
                for q in range(GS):
                    pltpu.async_copy(
                        table.at[src_v.at[0, pl.ds(q * QS, QS)]],
                        rows_v.at[0, pl.ds(q * QS, QS)], gsem)

                def body(b, carry, table=table):
                    j = lax.rem(b, 2)
                    jn = lax.rem(b + 1, 2)

                    @pl.when(b > 0)
                    def _():
                        pltpu.make_async_copy(
                            rows_v.at[jn],
                            agg_sp.at[dst_v.at[b - 1]], ssem).wait()

                    @pl.when(b + 1 < HB)
                    def _():
                        for q in range(GS):
                            pltpu.async_copy(
                                table.at[src_v.at[b + 1, pl.ds(q * QS, QS)]],
                                rows_v.at[jn, pl.ds(q * QS, QS)], gsem)

                    pltpu.make_async_copy(
                        table.at[src_v.at[b]], rows_v.at[j], gsem).wait()

                    pltpu.async_copy(
                        rows_v.at[j], agg_sp.at[dst_v.at[b]], ssem, add=True)
                    return carry

                lax.fori_loop(0, HB, body, None)
                pltpu.make_async_copy(
                    rows_v.at[(HB - 1) % 2],
                    agg_sp.at[dst_v.at[HB - 1]], ssem).wait()
            plsc.subcore_barrier()
            pltpu.sync_copy(
                agg_sp.at[pl.ds(s * ROWS_PER_TILE, ROWS_PER_TILE)],
                out_hbm.at[r, c, pl.ds(s * ROWS_PER_TILE, ROWS_PER_TILE)])

    return agg


_agg1 = _make_agg_kernel(1)
_agg2 = _make_agg_kernel(2)


def _norm(deg_blk):
    return lax.rsqrt(jnp.maximum(deg_blk, 1.0))


def _pre_body(x_ref, dout_ref, out_ref):
    xs = x_ref[...] * _norm(dout_ref[...])
    out_ref[0] = xs[:, :128]
    out_ref[1] = xs[:, 128:]


def _mm1_body(agg_ref, din_ref, dout_ref, w_ref, b_ref, out_ref):
    a = jnp.concatenate([agg_ref[0], agg_ref[1]], axis=1) * _norm(din_ref[...])
    h = jnp.dot(a, w_ref[...], preferred_element_type=jnp.float32) + b_ref[...]
    h = jnp.maximum(h, 0.0) * _norm(dout_ref[...])
    for j in range(4):
        out_ref[j] = h[:, j * 128:(j + 1) * 128]


def _mm2_body(agg_ref, din_ref, wc2_ref, bc2_ref, wl1_ref, bl1_ref, wo_ref,
              bo_ref, out_ref):
    a = jnp.concatenate([agg_ref[j] for j in range(4)], axis=1)
    a = a * _norm(din_ref[...])
    h = jnp.dot(a, wc2_ref[...], preferred_element_type=jnp.float32)
    h = jnp.maximum(h + bc2_ref[...], 0.0)
    h = jnp.dot(h, wl1_ref[...], preferred_element_type=jnp.float32)
    h = jnp.maximum(h + bl1_ref[...], 0.0)
    out_ref[...] = (jnp.dot(h, wo_ref[...], preferred_element_type=jnp.float32)
                    + bo_ref[...])


def _full(shape):
    return pl.BlockSpec(shape, lambda i: tuple(0 for _ in shape))


_pre_call = pl.pallas_call(
    _pre_body,
    grid=(GRID,),
    in_specs=[
        pl.BlockSpec((BN, 256), lambda i: (i, 0)),
        pl.BlockSpec((BN, 1), lambda i: (i, 0)),
    ],
    out_specs=pl.BlockSpec((2, BN, 128), lambda i: (0, i, 0)),
    out_shape=jax.ShapeDtypeStruct((2, NP, 128), jnp.float32),
)

_mm1_call = pl.pallas_call(
    _mm1_body,
    grid=(GRID,),
    in_specs=[
        pl.BlockSpec((2, BN, 128), lambda i: (0, i, 0)),
        pl.BlockSpec((BN, 1), lambda i: (i, 0)),
        pl.BlockSpec((BN, 1), lambda i: (i, 0)),
        _full((256, 512)),
        _full((1, 512)),
    ],
    out_specs=pl.BlockSpec((4, BN, 128), lambda i: (0, i, 0)),
    out_shape=jax.ShapeDtypeStruct((4, NP, 128), jnp.float32),
)

_mm2_call = pl.pallas_call(
    _mm2_body,
    grid=(GRID,),
    in_specs=[
        pl.BlockSpec((4, BN, 128), lambda i: (0, i, 0)),
        pl.BlockSpec((BN, 1), lambda i: (i, 0)),
        _full((512, 512)),
        _full((1, 512)),
        _full((512, 512)),
        _full((1, 512)),
        _full((512, 128)),
        _full((1, 128)),
    ],
    out_specs=pl.BlockSpec((BN, 128), lambda i: (i, 0)),
    out_shape=jax.ShapeDtypeStruct((NP, 128), jnp.float32),
)


def kernel(x, edge_index, Wc1, bc1, Wc2, bc2, Wl1, bl1, Wo, bo):
    src = edge_index[0].astype(jnp.int32)
    dst = edge_index[1].astype(jnp.int32)
    pad = EPAD - E
    src_g = jnp.concatenate([src, jnp.zeros((pad,), jnp.int32)])
    dst_p = jnp.concatenate([dst, jnp.full((pad,), NDUMP, jnp.int32)])
    src_d = jnp.concatenate([src, jnp.full((pad,), NDUMP, jnp.int32)])

    src_idx = jnp.stack([src_g, src_g + NP]).reshape(NC, NS, NBATCH, 128)
    dst_idx = dst_p.reshape(NS, NBATCH, 128)
    deg_idx = jnp.stack([src_d, dst_p]).reshape(NC, NS, NBATCH, 128)

    zeros_flat = jnp.zeros((ROWS_PER_TILE,), jnp.float32)
    zeros = jnp.zeros((ROWS_PER_TILE, 128), jnp.float32)
    ones = jnp.ones((128,), jnp.float32)

    degs = _deg_kernel(deg_idx, zeros_flat, ones)
    deg_out = degs[0].reshape(NP, 1)
    deg_in = degs[1].reshape(NP, 1)

    x_pad = jnp.pad(x, ((0, NP - N), (0, 0)))

    xs_parts = _pre_call(x_pad, deg_out)
    table1 = xs_parts.reshape(2 * NP, 128)
    agg1 = _agg1(table1, src_idx, dst_idx, zeros)
    h1s_parts = _mm1_call(agg1.reshape(NC, NP, 128), deg_in, deg_out,
                          Wc1, bc1.reshape(1, 512))

    tables2 = h1s_parts.reshape(2, 2 * NP, 128)
    agg2 = _agg2(tables2[0], tables2[1], src_idx, dst_idx, zeros)
    out = _mm2_call(agg2.reshape(4, NP, 128), deg_in,
                    Wc2, bc2.reshape(1, 512),
                    Wl1, bl1.reshape(1, 512),
                    Wo, bo.reshape(1, 128))
    return out[:N]
